# baseline (speedup 1.0000x reference)
import jax
import jax.numpy as jnp
from jax import lax
from jax.experimental import pallas as pl
from jax.experimental.pallas import tpu as pltpu
from jax.experimental.pallas import tpu_sc as plsc

_N = 1000
_F = 16
_B = 1024

_info = plsc.get_sparse_core_info()
_NC = _info.num_cores
_NS = _info.num_subcores
_NW = _NC * _NS


def _sc_body(wt_hbm, out_hbm, wt_v, plane, sem):
    wid = lax.axis_index("s") * _NC + lax.axis_index("c")
    pltpu.sync_copy(wt_hbm, wt_v)
    lane = jnp.full((16,), wid % 16, jnp.int32)
    base16 = (wid // 16) * 16

    def row(r, carry):
        n = wid + r * _NW

        @pl.when(n < _N)
        def _():
            for f in range(_F):
                v = wt_v[pl.ds(f * _N + base16 + r * _NW, 16)]
                s = lax.gather(
                    v,
                    lane[:, None],
                    lax.GatherDimensionNumbers(
                        offset_dims=(),
                        collapsed_slice_dims=(0,),
                        start_index_map=(0,),
                    ),
                    slice_sizes=(1,),
                    mode=lax.GatherScatterMode.PROMISE_IN_BOUNDS,
                )
                for j in range(_B // 16):
                    plane[pl.ds(f * _B + j * 16, 16)] = s
            copies = [
                pltpu.make_async_copy(
                    plane.at[pl.ds(f * _B, _B)], out_hbm.at[n, f], sem
                )
                for f in range(_F)
            ]
            for c in copies:
                c.start()
            for c in copies:
                c.wait()

        return carry

    lax.fori_loop(0, -(-_N // _NW), row, 0)


@jax.jit
def kernel(x, embed_weight):
    del x
    mesh = plsc.VectorSubcoreMesh(core_axis_name="c", subcore_axis_name="s")
    out_t = pl.kernel(
        _sc_body,
        out_type=jax.ShapeDtypeStruct((_N, _F, _B), jnp.float32),
        mesh=mesh,
        scratch_types=[
            pltpu.VMEM((_F * _N,), jnp.float32),
            pltpu.VMEM((_F * _B,), jnp.float32),
            pltpu.SemaphoreType.DMA,
        ],
    )(embed_weight.T.reshape(-1))
    return jnp.transpose(out_t, (2, 0, 1))


# speedup vs baseline: 3.0952x; 3.0952x over previous
import jax
import jax.numpy as jnp
from jax.experimental import pallas as pl
from jax.experimental.pallas import tpu as pltpu

_N = 1000
_F = 16
_B = 1024
_NB = 64
_G = -(-_N // _NB)


def _broadcast_body(w_ref, out_ref, wt_ref):
    i = pl.program_id(0)

    @pl.when(i == 0)
    def _():
        wt_ref[pl.ds(0, _N), :] = w_ref[...].T

    chunk = wt_ref[pl.ds(i * _NB, _NB), :]
    out_ref[...] = jnp.broadcast_to(chunk[:, :, None], (_NB, _F, _B))


@jax.jit
def kernel(x, embed_weight):
    del x
    out_t = pl.pallas_call(
        _broadcast_body,
        grid=(_G,),
        in_specs=[pl.BlockSpec((_F, _N), lambda i: (0, 0))],
        out_specs=pl.BlockSpec((_NB, _F, _B), lambda i: (i, 0, 0)),
        out_shape=jax.ShapeDtypeStruct((_N, _F, _B), jnp.float32),
        scratch_shapes=[pltpu.VMEM((1024, _F), jnp.float32)],
    )(embed_weight.T)
    return jnp.transpose(out_t, (2, 0, 1))
